# baseline (speedup 1.0000x reference)
import functools

import jax
import jax.numpy as jnp
from jax import lax
from jax.experimental import pallas as pl
from jax.experimental.pallas import tpu as pltpu
from jax.experimental.pallas import tpu_sc as plsc

N = 10000
D = 128
E = 320000
NC = 2
NS = 16
NW = NC * NS
CK = 80
EPW = E // NW
NCHUNK = EPW // CK
NP = 10240
RPT = NP // NS
ZROWS = 128
DEGP = NP
DPT = DEGP // NS
NBUF = 2

_MESH = plsc.VectorSubcoreMesh(
    core_axis_name="c", subcore_axis_name="s", num_cores=NC, num_subcores=NS)


def _deg_body(dst_hbm, out_hbm, dvs, ones_v, zb_v, deg_sh, semds, semss):
    c = lax.axis_index("c")
    s = lax.axis_index("s")
    ebase = (c * NS + s) * EPW
    for k in range(NBUF):
        pltpu.async_copy(dst_hbm.at[pl.ds(ebase + k * CK, CK)],
                         dvs[k], semds[k])
    ones16 = jnp.ones((16,), jnp.float32)
    zero16 = jnp.zeros((16,), jnp.float32)
    for j in range(CK // 16):
        ones_v[pl.ds(j * 16, 16)] = ones16
    def zfill(i, _):
        zb_v[pl.ds(i * 16, 16)] = zero16
        return 0
    lax.fori_loop(0, DPT // 16, zfill, 0)
    pltpu.sync_copy(zb_v, deg_sh.at[pl.ds(s * DPT, DPT)])
    plsc.subcore_barrier()

    def body(i, _):
        j0 = NBUF * i
        for k in range(NBUF):
            pltpu.make_async_copy(dst_hbm.at[pl.ds(0, CK)],
                                  dvs[k], semds[k]).wait()
            pltpu.async_copy(ones_v, deg_sh.at[dvs[k]], semss[k], add=True)
        for k in range(NBUF):
            jn = j0 + NBUF + k
            pltpu.make_async_copy(ones_v, deg_sh.at[dvs[k]],
                                  semss[k]).wait()
            @pl.when(jn < NCHUNK)
            def _():
                pltpu.async_copy(dst_hbm.at[pl.ds(ebase + jn * CK, CK)],
                                 dvs[k], semds[k])
        return 0
    lax.fori_loop(0, (NCHUNK - 1) // NBUF, body, 0)
    pltpu.make_async_copy(dst_hbm.at[pl.ds(0, CK)], dvs[0], semds[0]).wait()
    pltpu.sync_copy(ones_v, deg_sh.at[dvs[0]], add=True)
    plsc.subcore_barrier()
    pltpu.sync_copy(deg_sh.at[pl.ds(s * DPT, DPT)],
                    out_hbm.at[pl.ds(c * DEGP + s * DPT, DPT)])


_deg_call = pl.kernel(
    _deg_body,
    out_type=jax.ShapeDtypeStruct((NC * DEGP,), jnp.float32),
    mesh=_MESH,
    scratch_types=[
        [pltpu.VMEM((CK,), jnp.int32)] * NBUF,
        pltpu.VMEM((CK,), jnp.float32),
        pltpu.VMEM((DPT,), jnp.float32),
        pltpu.VMEM_SHARED((DEGP,), jnp.float32),
        [pltpu.SemaphoreType.DMA] * NBUF,
        [pltpu.SemaphoreType.DMA] * NBUF,
    ],
)


def _scatter_body(g_hbm, src_hbm, dst_hbm, out_hbm,
                  sslab_v, dvs, rowss, zb_v, s_sh,
                  semi, semds, semgs, semss):
    c = lax.axis_index("c")
    s = lax.axis_index("s")
    w = c * NS + s
    ebase = w * EPW
    cpi1 = pltpu.async_copy(src_hbm.at[pl.ds(ebase, EPW)], sslab_v, semi)
    zero16 = jnp.zeros((16,), jnp.float32)
    def zfill(i, _):
        for j in range(D // 16):
            zb_v[i, pl.ds(j * 16, 16)] = zero16
        return 0
    lax.fori_loop(0, ZROWS, zfill, 0)
    for r in range(RPT // ZROWS):
        pltpu.sync_copy(zb_v, s_sh.at[pl.ds(s * RPT + r * ZROWS, ZROWS)])
    for k in range(NBUF):
        pltpu.async_copy(dst_hbm.at[pl.ds(ebase + k * CK, CK)],
                         dvs[k], semds[k])
    cpi1.wait()
    for k in range(NBUF):
        pltpu.async_copy(g_hbm.at[sslab_v.at[pl.ds(k * CK, CK)]],
                         rowss[k], semgs[k])
    plsc.subcore_barrier()

    def body(i, _):
        j0 = NBUF * i
        for k in range(NBUF):
            pltpu.make_async_copy(g_hbm.at[sslab_v.at[pl.ds(0, CK)]],
                                  rowss[k], semgs[k]).wait()
            pltpu.make_async_copy(dst_hbm.at[pl.ds(0, CK)],
                                  dvs[k], semds[k]).wait()
            pltpu.async_copy(rowss[k], s_sh.at[dvs[k]], semss[k], add=True)
        for k in range(NBUF):
            jn = j0 + NBUF + k
            pltpu.make_async_copy(rowss[k], s_sh.at[dvs[k]],
                                  semss[k]).wait()
            @pl.when(jn < NCHUNK)
            def _():
                pltpu.async_copy(dst_hbm.at[pl.ds(ebase + jn * CK, CK)],
                                 dvs[k], semds[k])
                pltpu.async_copy(g_hbm.at[sslab_v.at[pl.ds(jn * CK, CK)]],
                                 rowss[k], semgs[k])
        return 0
    lax.fori_loop(0, (NCHUNK - 1) // NBUF, body, 0)
    pltpu.make_async_copy(g_hbm.at[sslab_v.at[pl.ds(0, CK)]],
                          rowss[0], semgs[0]).wait()
    pltpu.make_async_copy(dst_hbm.at[pl.ds(0, CK)], dvs[0], semds[0]).wait()
    pltpu.sync_copy(rowss[0], s_sh.at[dvs[0]], add=True)
    plsc.subcore_barrier()
    pltpu.sync_copy(s_sh.at[pl.ds(s * RPT, RPT)],
                    out_hbm.at[pl.ds(c * NP + s * RPT, RPT)])


_scatter_call = pl.kernel(
    _scatter_body,
    out_type=jax.ShapeDtypeStruct((NC * NP, D), jnp.float32),
    mesh=_MESH,
    scratch_types=[
        pltpu.VMEM((EPW,), jnp.int32),
        [pltpu.VMEM((CK,), jnp.int32)] * NBUF,
        [pltpu.VMEM((CK, D), jnp.float32)] * NBUF,
        pltpu.VMEM((ZROWS, D), jnp.float32),
        pltpu.VMEM_SHARED((NP, D), jnp.float32),
        pltpu.SemaphoreType.DMA,
        [pltpu.SemaphoreType.DMA] * NBUF,
        [pltpu.SemaphoreType.DMA] * NBUF,
        [pltpu.SemaphoreType.DMA] * NBUF,
    ],
)


_TB = 1000


def _tc1_body(dinv_ref, x_ref, w_ref, o_ref):
    o_ref[...] = jnp.dot(dinv_ref[...] * x_ref[...], w_ref[...],
                         preferred_element_type=jnp.float32)


_tc1_call = pl.pallas_call(
    _tc1_body,
    grid=(N // _TB,),
    in_specs=[
        pl.BlockSpec((_TB, 1), lambda i: (i, 0)),
        pl.BlockSpec((_TB, D), lambda i: (i, 0)),
        pl.BlockSpec((D, D), lambda i: (0, 0)),
    ],
    out_specs=pl.BlockSpec((_TB, D), lambda i: (i, 0)),
    out_shape=jax.ShapeDtypeStruct((N, D), jnp.float32),
)


def _tc2_body(sp_ref, g_ref, dinv_ref, b_ref, w_ref, o_ref):
    ssum = sp_ref[0] + sp_ref[1]
    h = jnp.maximum(dinv_ref[...] * (ssum + g_ref[...]) + b_ref[...], 0.0)
    o_ref[...] = jnp.dot(dinv_ref[...] * h, w_ref[...],
                         preferred_element_type=jnp.float32)


_tc2_call = pl.pallas_call(
    _tc2_body,
    grid=(N // _TB,),
    in_specs=[
        pl.BlockSpec((2, _TB, D), lambda i: (0, i, 0)),
        pl.BlockSpec((_TB, D), lambda i: (i, 0)),
        pl.BlockSpec((_TB, 1), lambda i: (i, 0)),
        pl.BlockSpec((1, D), lambda i: (0, 0)),
        pl.BlockSpec((D, D), lambda i: (0, 0)),
    ],
    out_specs=pl.BlockSpec((_TB, D), lambda i: (i, 0)),
    out_shape=jax.ShapeDtypeStruct((N, D), jnp.float32),
)


def _tc3_body(sp_ref, g_ref, dinv_ref, b_ref, o_ref):
    ssum = sp_ref[0] + sp_ref[1]
    o_ref[...] = jnp.maximum(
        dinv_ref[...] * (ssum + g_ref[...]) + b_ref[...], 0.0)


_tc3_call = pl.pallas_call(
    _tc3_body,
    grid=(N // _TB,),
    in_specs=[
        pl.BlockSpec((2, _TB, D), lambda i: (0, i, 0)),
        pl.BlockSpec((_TB, D), lambda i: (i, 0)),
        pl.BlockSpec((_TB, 1), lambda i: (i, 0)),
        pl.BlockSpec((1, D), lambda i: (0, 0)),
    ],
    out_specs=pl.BlockSpec((_TB, D), lambda i: (i, 0)),
    out_shape=jax.ShapeDtypeStruct((N, D), jnp.float32),
)


def kernel(x, edge_index, W1, b1, W2, b2):
    ei = edge_index.astype(jnp.int32)
    src = ei[0]
    dst = ei[1]

    degp = _deg_call(dst)
    deg = 1.0 + degp[:N] + degp[DEGP:DEGP + N]
    dinv = lax.rsqrt(deg)[:, None]
    b1r = b1[None, :]
    b2r = b2[None, :]

    g1 = _tc1_call(dinv, x, W1)
    s1 = _scatter_call(g1, src, dst).reshape(NC, NP, D)
    g2 = _tc2_call(s1, g1, dinv, b1r, W2)
    s2 = _scatter_call(g2, src, dst).reshape(NC, NP, D)
    return _tc3_call(s2, g2, dinv, b2r)


# speedup vs baseline: 1.2134x; 1.2134x over previous
import functools

import jax
import jax.numpy as jnp
from jax import lax
from jax.experimental import pallas as pl
from jax.experimental.pallas import tpu as pltpu
from jax.experimental.pallas import tpu_sc as plsc

N = 10000
D = 128
E = 320000
NC = 2
NS = 16
NW = NC * NS
CK = 80
EPW = E // NW
NCHUNK = EPW // CK
NP = 10240
RPT = NP // NS
ZROWS = 128
DEGP = NP
DPT = DEGP // NS
NBUF = 2

_MESH = plsc.VectorSubcoreMesh(
    core_axis_name="c", subcore_axis_name="s", num_cores=NC, num_subcores=NS)


def _deg_body(dst_hbm, out_hbm, dvs, ones_v, zb_v, deg_sh, semds, semss):
    c = lax.axis_index("c")
    s = lax.axis_index("s")
    ebase = (c * NS + s) * EPW
    for k in range(NBUF):
        pltpu.async_copy(dst_hbm.at[pl.ds(ebase + k * CK, CK)],
                         dvs[k], semds[k])
    ones16 = jnp.ones((16,), jnp.float32)
    zero16 = jnp.zeros((16,), jnp.float32)
    for j in range(CK // 16):
        ones_v[pl.ds(j * 16, 16)] = ones16
    def zfill(i, _):
        zb_v[pl.ds(i * 16, 16)] = zero16
        return 0
    lax.fori_loop(0, DPT // 16, zfill, 0)
    pltpu.sync_copy(zb_v, deg_sh.at[pl.ds(s * DPT, DPT)])
    plsc.subcore_barrier()

    def body(i, _):
        j0 = NBUF * i
        for k in range(NBUF):
            pltpu.make_async_copy(dst_hbm.at[pl.ds(0, CK)],
                                  dvs[k], semds[k]).wait()
            pltpu.async_copy(ones_v, deg_sh.at[dvs[k]], semss[k], add=True)
        for k in range(NBUF):
            jn = j0 + NBUF + k
            pltpu.make_async_copy(ones_v, deg_sh.at[dvs[k]],
                                  semss[k]).wait()
            @pl.when(jn < NCHUNK)
            def _():
                pltpu.async_copy(dst_hbm.at[pl.ds(ebase + jn * CK, CK)],
                                 dvs[k], semds[k])
        return 0
    lax.fori_loop(0, (NCHUNK - 1) // NBUF, body, 0)
    pltpu.make_async_copy(dst_hbm.at[pl.ds(0, CK)], dvs[0], semds[0]).wait()
    pltpu.sync_copy(ones_v, deg_sh.at[dvs[0]], add=True)
    plsc.subcore_barrier()
    pltpu.sync_copy(deg_sh.at[pl.ds(s * DPT, DPT)],
                    out_hbm.at[pl.ds(c * DEGP + s * DPT, DPT)])


_deg_call = pl.kernel(
    _deg_body,
    out_type=jax.ShapeDtypeStruct((NC * DEGP,), jnp.float32),
    mesh=_MESH,
    scratch_types=[
        [pltpu.VMEM((CK,), jnp.int32)] * NBUF,
        pltpu.VMEM((CK,), jnp.float32),
        pltpu.VMEM((DPT,), jnp.float32),
        pltpu.VMEM_SHARED((DEGP,), jnp.float32),
        [pltpu.SemaphoreType.DMA] * NBUF,
        [pltpu.SemaphoreType.DMA] * NBUF,
    ],
)


def _scatter_body(g_hbm, src_hbm, dst_hbm, out_hbm,
                  sslab_v, dvs, rowss, zb_v, s_sh,
                  semi, semds, semgs, semss):
    c = lax.axis_index("c")
    s = lax.axis_index("s")
    w = c * NS + s
    ebase = w * EPW
    cpi1 = pltpu.async_copy(src_hbm.at[pl.ds(ebase, EPW)], sslab_v, semi)
    zero16 = jnp.zeros((16,), jnp.float32)
    def zfill(i, _):
        for j in range(D // 16):
            zb_v[i, pl.ds(j * 16, 16)] = zero16
        return 0
    lax.fori_loop(0, ZROWS, zfill, 0)
    for r in range(RPT // ZROWS):
        pltpu.sync_copy(zb_v, s_sh.at[pl.ds(s * RPT + r * ZROWS, ZROWS)])
    for k in range(NBUF):
        pltpu.async_copy(dst_hbm.at[pl.ds(ebase + k * CK, CK)],
                         dvs[k], semds[k])
    cpi1.wait()
    for k in range(NBUF):
        pltpu.async_copy(g_hbm.at[sslab_v.at[pl.ds(k * CK, CK)]],
                         rowss[k], semgs[k])
    plsc.subcore_barrier()

    def body(i, _):
        j0 = NBUF * i
        for k in range(NBUF):
            jn = j0 + NBUF + k
            pltpu.make_async_copy(g_hbm.at[sslab_v.at[pl.ds(0, CK)]],
                                  rowss[k], semgs[k]).wait()
            pltpu.make_async_copy(dst_hbm.at[pl.ds(0, CK)],
                                  dvs[k], semds[k]).wait()
            pltpu.sync_copy(rowss[k], s_sh.at[dvs[k]], add=True)
            @pl.when(jn < NCHUNK)
            def _():
                pltpu.async_copy(dst_hbm.at[pl.ds(ebase + jn * CK, CK)],
                                 dvs[k], semds[k])
                pltpu.async_copy(g_hbm.at[sslab_v.at[pl.ds(jn * CK, CK)]],
                                 rowss[k], semgs[k])
        return 0
    lax.fori_loop(0, (NCHUNK - 1) // NBUF, body, 0)
    pltpu.make_async_copy(g_hbm.at[sslab_v.at[pl.ds(0, CK)]],
                          rowss[0], semgs[0]).wait()
    pltpu.make_async_copy(dst_hbm.at[pl.ds(0, CK)], dvs[0], semds[0]).wait()
    pltpu.sync_copy(rowss[0], s_sh.at[dvs[0]], add=True)
    plsc.subcore_barrier()
    pltpu.sync_copy(s_sh.at[pl.ds(s * RPT, RPT)],
                    out_hbm.at[pl.ds(c * NP + s * RPT, RPT)])


_scatter_call = pl.kernel(
    _scatter_body,
    out_type=jax.ShapeDtypeStruct((NC * NP, D), jnp.float32),
    mesh=_MESH,
    scratch_types=[
        pltpu.VMEM((EPW,), jnp.int32),
        [pltpu.VMEM((CK,), jnp.int32)] * NBUF,
        [pltpu.VMEM((CK, D), jnp.float32)] * NBUF,
        pltpu.VMEM((ZROWS, D), jnp.float32),
        pltpu.VMEM_SHARED((NP, D), jnp.float32),
        pltpu.SemaphoreType.DMA,
        [pltpu.SemaphoreType.DMA] * NBUF,
        [pltpu.SemaphoreType.DMA] * NBUF,
        [pltpu.SemaphoreType.DMA] * NBUF,
    ],
)


_TB = 1000


def _tc1_body(dinv_ref, x_ref, w_ref, o_ref):
    o_ref[...] = jnp.dot(dinv_ref[...] * x_ref[...], w_ref[...],
                         preferred_element_type=jnp.float32)


_tc1_call = pl.pallas_call(
    _tc1_body,
    grid=(N // _TB,),
    in_specs=[
        pl.BlockSpec((_TB, 1), lambda i: (i, 0)),
        pl.BlockSpec((_TB, D), lambda i: (i, 0)),
        pl.BlockSpec((D, D), lambda i: (0, 0)),
    ],
    out_specs=pl.BlockSpec((_TB, D), lambda i: (i, 0)),
    out_shape=jax.ShapeDtypeStruct((N, D), jnp.float32),
)


def _tc2_body(sp_ref, g_ref, dinv_ref, b_ref, w_ref, o_ref):
    ssum = sp_ref[0] + sp_ref[1]
    h = jnp.maximum(dinv_ref[...] * (ssum + g_ref[...]) + b_ref[...], 0.0)
    o_ref[...] = jnp.dot(dinv_ref[...] * h, w_ref[...],
                         preferred_element_type=jnp.float32)


_tc2_call = pl.pallas_call(
    _tc2_body,
    grid=(N // _TB,),
    in_specs=[
        pl.BlockSpec((2, _TB, D), lambda i: (0, i, 0)),
        pl.BlockSpec((_TB, D), lambda i: (i, 0)),
        pl.BlockSpec((_TB, 1), lambda i: (i, 0)),
        pl.BlockSpec((1, D), lambda i: (0, 0)),
        pl.BlockSpec((D, D), lambda i: (0, 0)),
    ],
    out_specs=pl.BlockSpec((_TB, D), lambda i: (i, 0)),
    out_shape=jax.ShapeDtypeStruct((N, D), jnp.float32),
)


def _tc3_body(sp_ref, g_ref, dinv_ref, b_ref, o_ref):
    ssum = sp_ref[0] + sp_ref[1]
    o_ref[...] = jnp.maximum(
        dinv_ref[...] * (ssum + g_ref[...]) + b_ref[...], 0.0)


_tc3_call = pl.pallas_call(
    _tc3_body,
    grid=(N // _TB,),
    in_specs=[
        pl.BlockSpec((2, _TB, D), lambda i: (0, i, 0)),
        pl.BlockSpec((_TB, D), lambda i: (i, 0)),
        pl.BlockSpec((_TB, 1), lambda i: (i, 0)),
        pl.BlockSpec((1, D), lambda i: (0, 0)),
    ],
    out_specs=pl.BlockSpec((_TB, D), lambda i: (i, 0)),
    out_shape=jax.ShapeDtypeStruct((N, D), jnp.float32),
)


def kernel(x, edge_index, W1, b1, W2, b2):
    ei = edge_index.astype(jnp.int32)
    src = ei[0]
    dst = ei[1]

    degp = _deg_call(dst)
    deg = 1.0 + degp[:N] + degp[DEGP:DEGP + N]
    dinv = lax.rsqrt(deg)[:, None]
    b1r = b1[None, :]
    b2r = b2[None, :]

    g1 = _tc1_call(dinv, x, W1)
    s1 = _scatter_call(g1, src, dst).reshape(NC, NP, D)
    g2 = _tc2_call(s1, g1, dinv, b1r, W2)
    s2 = _scatter_call(g2, src, dst).reshape(NC, NP, D)
    return _tc3_call(s2, g2, dinv, b2r)


# speedup vs baseline: 1.2980x; 1.0697x over previous
import functools

import jax
import jax.numpy as jnp
from jax import lax
from jax.experimental import pallas as pl
from jax.experimental.pallas import tpu as pltpu
from jax.experimental.pallas import tpu_sc as plsc

N = 10000
D = 128
E = 320000
NC = 2
NS = 16
NW = NC * NS
CK = 80
EPW = E // NW
NCHUNK = EPW // CK
NP = 10240
RPT = NP // NS
ZROWS = 128
DEGP = NP
DPT = DEGP // NS
NBUF = 2
NBUFD = 4

_MESH = plsc.VectorSubcoreMesh(
    core_axis_name="c", subcore_axis_name="s", num_cores=NC, num_subcores=NS)


def _deg_body(dst_hbm, out_hbm, dvs, ones_v, zb_v, deg_sh, semds, semss):
    c = lax.axis_index("c")
    s = lax.axis_index("s")
    ebase = (c * NS + s) * EPW
    for k in range(NBUFD):
        pltpu.async_copy(dst_hbm.at[pl.ds(ebase + k * CK, CK)],
                         dvs[k], semds[k])
    ones16 = jnp.ones((16,), jnp.float32)
    zero16 = jnp.zeros((16,), jnp.float32)
    for j in range(CK // 16):
        ones_v[pl.ds(j * 16, 16)] = ones16
    def zfill(i, _):
        zb_v[pl.ds(i * 16, 16)] = zero16
        return 0
    lax.fori_loop(0, DPT // 16, zfill, 0)
    pltpu.sync_copy(zb_v, deg_sh.at[pl.ds(s * DPT, DPT)])
    plsc.subcore_barrier()

    def body(i, _):
        j0 = NBUFD * i
        for k in range(NBUFD):
            pltpu.make_async_copy(dst_hbm.at[pl.ds(0, CK)],
                                  dvs[k], semds[k]).wait()
            pltpu.async_copy(ones_v, deg_sh.at[dvs[k]], semss[k], add=True)
        for k in range(NBUFD):
            jn = j0 + NBUFD + k
            pltpu.make_async_copy(ones_v, deg_sh.at[dvs[k]],
                                  semss[k]).wait()
            @pl.when(jn < NCHUNK)
            def _():
                pltpu.async_copy(dst_hbm.at[pl.ds(ebase + jn * CK, CK)],
                                 dvs[k], semds[k])
        return 0
    lax.fori_loop(0, (NCHUNK - 1) // NBUFD, body, 0)
    pltpu.make_async_copy(dst_hbm.at[pl.ds(0, CK)], dvs[0], semds[0]).wait()
    pltpu.sync_copy(ones_v, deg_sh.at[dvs[0]], add=True)
    plsc.subcore_barrier()
    pltpu.sync_copy(deg_sh.at[pl.ds(s * DPT, DPT)],
                    out_hbm.at[pl.ds(c * DEGP + s * DPT, DPT)])


_deg_call = pl.kernel(
    _deg_body,
    out_type=jax.ShapeDtypeStruct((NC * DEGP,), jnp.float32),
    mesh=_MESH,
    scratch_types=[
        [pltpu.VMEM((CK,), jnp.int32)] * NBUFD,
        pltpu.VMEM((CK,), jnp.float32),
        pltpu.VMEM((DPT,), jnp.float32),
        pltpu.VMEM_SHARED((DEGP,), jnp.float32),
        [pltpu.SemaphoreType.DMA] * NBUFD,
        [pltpu.SemaphoreType.DMA] * NBUFD,
    ],
)


def _scatter_body(g_hbm, src_hbm, dst_hbm, out_hbm,
                  sslab_v, dvs, rowss, zb_v, s_sh,
                  semi, semds, semgs, semss):
    c = lax.axis_index("c")
    s = lax.axis_index("s")
    w = c * NS + s
    ebase = w * EPW
    cpi1 = pltpu.async_copy(src_hbm.at[pl.ds(ebase, EPW)], sslab_v, semi)
    zero16 = jnp.zeros((16,), jnp.float32)
    def zfill(i, _):
        for j in range(D // 16):
            zb_v[i, pl.ds(j * 16, 16)] = zero16
        return 0
    lax.fori_loop(0, ZROWS, zfill, 0)
    for r in range(RPT // ZROWS):
        pltpu.sync_copy(zb_v, s_sh.at[pl.ds(s * RPT + r * ZROWS, ZROWS)])
    for k in range(NBUF):
        pltpu.async_copy(dst_hbm.at[pl.ds(ebase + k * CK, CK)],
                         dvs[k], semds[k])
    cpi1.wait()
    for k in range(NBUF):
        pltpu.async_copy(g_hbm.at[sslab_v.at[pl.ds(k * CK, CK)]],
                         rowss[k], semgs[k])
    plsc.subcore_barrier()

    def body(i, _):
        j0 = NBUF * i
        for k in range(NBUF):
            jn = j0 + NBUF + k
            pltpu.make_async_copy(g_hbm.at[sslab_v.at[pl.ds(0, CK)]],
                                  rowss[k], semgs[k]).wait()
            pltpu.make_async_copy(dst_hbm.at[pl.ds(0, CK)],
                                  dvs[k], semds[k]).wait()
            pltpu.sync_copy(rowss[k], s_sh.at[dvs[k]], add=True)
            @pl.when(jn < NCHUNK)
            def _():
                pltpu.async_copy(dst_hbm.at[pl.ds(ebase + jn * CK, CK)],
                                 dvs[k], semds[k])
                pltpu.async_copy(g_hbm.at[sslab_v.at[pl.ds(jn * CK, CK)]],
                                 rowss[k], semgs[k])
        return 0
    lax.fori_loop(0, (NCHUNK - 1) // NBUF, body, 0)
    pltpu.make_async_copy(g_hbm.at[sslab_v.at[pl.ds(0, CK)]],
                          rowss[0], semgs[0]).wait()
    pltpu.make_async_copy(dst_hbm.at[pl.ds(0, CK)], dvs[0], semds[0]).wait()
    pltpu.sync_copy(rowss[0], s_sh.at[dvs[0]], add=True)
    plsc.subcore_barrier()
    pltpu.sync_copy(s_sh.at[pl.ds(s * RPT, RPT)],
                    out_hbm.at[pl.ds(c * NP + s * RPT, RPT)])


_scatter_call = pl.kernel(
    _scatter_body,
    out_type=jax.ShapeDtypeStruct((NC * NP, D), jnp.float32),
    mesh=_MESH,
    scratch_types=[
        pltpu.VMEM((EPW,), jnp.int32),
        [pltpu.VMEM((CK,), jnp.int32)] * NBUF,
        [pltpu.VMEM((CK, D), jnp.float32)] * NBUF,
        pltpu.VMEM((ZROWS, D), jnp.float32),
        pltpu.VMEM_SHARED((NP, D), jnp.float32),
        pltpu.SemaphoreType.DMA,
        [pltpu.SemaphoreType.DMA] * NBUF,
        [pltpu.SemaphoreType.DMA] * NBUF,
        [pltpu.SemaphoreType.DMA] * NBUF,
    ],
)


_TB = 1000


def _tc1_body(x_ref, w_ref, o_ref):
    o_ref[...] = jnp.dot(x_ref[...], w_ref[...],
                         preferred_element_type=jnp.float32)


_tc1_call = pl.pallas_call(
    _tc1_body,
    grid=(N // _TB,),
    in_specs=[
        pl.BlockSpec((_TB, D), lambda i: (i, 0)),
        pl.BlockSpec((D, D), lambda i: (0, 0)),
    ],
    out_specs=pl.BlockSpec((_TB, D), lambda i: (i, 0)),
    out_shape=jax.ShapeDtypeStruct((N, D), jnp.float32),
)


def _tc2_body(sp_ref, g_ref, dinv_ref, b_ref, w_ref, o_ref):
    ssum = sp_ref[0] + sp_ref[1]
    h = jnp.maximum(dinv_ref[...] * (ssum + g_ref[...]) + b_ref[...], 0.0)
    o_ref[...] = jnp.dot(dinv_ref[...] * h, w_ref[...],
                         preferred_element_type=jnp.float32)


_tc2_call = pl.pallas_call(
    _tc2_body,
    grid=(N // _TB,),
    in_specs=[
        pl.BlockSpec((2, _TB, D), lambda i: (0, i, 0)),
        pl.BlockSpec((_TB, D), lambda i: (i, 0)),
        pl.BlockSpec((_TB, 1), lambda i: (i, 0)),
        pl.BlockSpec((1, D), lambda i: (0, 0)),
        pl.BlockSpec((D, D), lambda i: (0, 0)),
    ],
    out_specs=pl.BlockSpec((_TB, D), lambda i: (i, 0)),
    out_shape=jax.ShapeDtypeStruct((N, D), jnp.float32),
)


def _tc3_body(sp_ref, g_ref, dinv_ref, b_ref, o_ref):
    ssum = sp_ref[0] + sp_ref[1]
    o_ref[...] = jnp.maximum(
        dinv_ref[...] * (ssum + g_ref[...]) + b_ref[...], 0.0)


_tc3_call = pl.pallas_call(
    _tc3_body,
    grid=(N // _TB,),
    in_specs=[
        pl.BlockSpec((2, _TB, D), lambda i: (0, i, 0)),
        pl.BlockSpec((_TB, D), lambda i: (i, 0)),
        pl.BlockSpec((_TB, 1), lambda i: (i, 0)),
        pl.BlockSpec((1, D), lambda i: (0, 0)),
    ],
    out_specs=pl.BlockSpec((_TB, D), lambda i: (i, 0)),
    out_shape=jax.ShapeDtypeStruct((N, D), jnp.float32),
)


def kernel(x, edge_index, W1, b1, W2, b2):
    ei = edge_index.astype(jnp.int32)
    src = ei[0]
    dst = ei[1]

    degp = _deg_call(dst)
    u1 = _tc1_call(x, W1)
    deg = 1.0 + degp[:N] + degp[DEGP:DEGP + N]
    dinv = lax.rsqrt(deg)[:, None]
    b1r = b1[None, :]
    b2r = b2[None, :]

    g1 = dinv * u1
    s1 = _scatter_call(g1, src, dst).reshape(NC, NP, D)
    g2 = _tc2_call(s1, g1, dinv, b1r, W2)
    s2 = _scatter_call(g2, src, dst).reshape(NC, NP, D)
    return _tc3_call(s2, g2, dinv, b2r)


# speedup vs baseline: 1.3001x; 1.0016x over previous
import jax
import jax.numpy as jnp
from jax import lax
from jax.experimental import pallas as pl
from jax.experimental.pallas import tpu as pltpu
from jax.experimental.pallas import tpu_sc as plsc

N = 10000
D = 128
E = 320000
NC = 2
NS = 16
NW = NC * NS
CK = 80
EPW = E // NW
NCHUNK = EPW // CK
NP = 10240
RPT = NP // NS
ZROWS = 128
DEGP = NP
DPT = DEGP // NS
NBUF = 2
NBUFD = 4

_MESH = plsc.VectorSubcoreMesh(
    core_axis_name="c", subcore_axis_name="s", num_cores=NC, num_subcores=NS)


def _deg_body(dst_hbm, out_hbm, dvs, ones_v, zb_v, deg_sh, semds, semss):
    c = lax.axis_index("c")
    s = lax.axis_index("s")
    ebase = (c * NS + s) * EPW
    for k in range(NBUFD):
        pltpu.async_copy(dst_hbm.at[pl.ds(ebase + k * CK, CK)],
                         dvs[k], semds[k])
    ones16 = jnp.ones((16,), jnp.float32)
    zero16 = jnp.zeros((16,), jnp.float32)
    for j in range(CK // 16):
        ones_v[pl.ds(j * 16, 16)] = ones16
    def zfill(i, _):
        zb_v[pl.ds(i * 16, 16)] = zero16
        return 0
    lax.fori_loop(0, DPT // 16, zfill, 0)
    pltpu.sync_copy(zb_v, deg_sh.at[pl.ds(s * DPT, DPT)])
    plsc.subcore_barrier()

    def body(i, _):
        j0 = NBUFD * i
        for k in range(NBUFD):
            pltpu.make_async_copy(dst_hbm.at[pl.ds(0, CK)],
                                  dvs[k], semds[k]).wait()
            pltpu.async_copy(ones_v, deg_sh.at[dvs[k]], semss[k], add=True)
        for k in range(NBUFD):
            jn = j0 + NBUFD + k
            pltpu.make_async_copy(ones_v, deg_sh.at[dvs[k]],
                                  semss[k]).wait()
            @pl.when(jn < NCHUNK)
            def _():
                pltpu.async_copy(dst_hbm.at[pl.ds(ebase + jn * CK, CK)],
                                 dvs[k], semds[k])
        return 0
    lax.fori_loop(0, (NCHUNK - 1) // NBUFD, body, 0)
    pltpu.make_async_copy(dst_hbm.at[pl.ds(0, CK)], dvs[0], semds[0]).wait()
    pltpu.sync_copy(ones_v, deg_sh.at[dvs[0]], add=True)
    plsc.subcore_barrier()
    pltpu.sync_copy(deg_sh.at[pl.ds(s * DPT, DPT)],
                    out_hbm.at[pl.ds(c * DEGP + s * DPT, DPT)])


_deg_call = pl.kernel(
    _deg_body,
    out_type=jax.ShapeDtypeStruct((NC * DEGP,), jnp.float32),
    mesh=_MESH,
    scratch_types=[
        [pltpu.VMEM((CK,), jnp.int32)] * NBUFD,
        pltpu.VMEM((CK,), jnp.float32),
        pltpu.VMEM((DPT,), jnp.float32),
        pltpu.VMEM_SHARED((DEGP,), jnp.float32),
        [pltpu.SemaphoreType.DMA] * NBUFD,
        [pltpu.SemaphoreType.DMA] * NBUFD,
    ],
)


def _scatter_body(g_hbm, src_hbm, dst_hbm, out_hbm,
                  sslab_v, dvs, rowss, zb_v, s_sh,
                  semi, semds, semgs):
    c = lax.axis_index("c")
    s = lax.axis_index("s")
    w = c * NS + s
    ebase = w * EPW
    cpi1 = pltpu.async_copy(src_hbm.at[pl.ds(ebase, EPW)], sslab_v, semi)
    zero16 = jnp.zeros((16,), jnp.float32)
    def zfill(i, _):
        for j in range(D // 16):
            zb_v[i, pl.ds(j * 16, 16)] = zero16
        return 0
    lax.fori_loop(0, ZROWS, zfill, 0)
    for r in range(RPT // ZROWS):
        pltpu.sync_copy(zb_v, s_sh.at[pl.ds(s * RPT + r * ZROWS, ZROWS)])
    for k in range(NBUF):
        pltpu.async_copy(dst_hbm.at[pl.ds(ebase + k * CK, CK)],
                         dvs[k], semds[k])
    cpi1.wait()
    for k in range(NBUF):
        pltpu.async_copy(g_hbm.at[sslab_v.at[pl.ds(k * CK, CK)]],
                         rowss[k], semgs[k])
    plsc.subcore_barrier()

    def body(i, _):
        j0 = NBUF * i
        for k in range(NBUF):
            jn = j0 + NBUF + k
            pltpu.make_async_copy(g_hbm.at[sslab_v.at[pl.ds(0, CK)]],
                                  rowss[k], semgs[k]).wait()
            pltpu.make_async_copy(dst_hbm.at[pl.ds(0, CK)],
                                  dvs[k], semds[k]).wait()
            pltpu.sync_copy(rowss[k], s_sh.at[dvs[k]], add=True)
            @pl.when(jn < NCHUNK)
            def _():
                pltpu.async_copy(dst_hbm.at[pl.ds(ebase + jn * CK, CK)],
                                 dvs[k], semds[k])
                pltpu.async_copy(g_hbm.at[sslab_v.at[pl.ds(jn * CK, CK)]],
                                 rowss[k], semgs[k])
        return 0
    lax.fori_loop(0, (NCHUNK - 1) // NBUF, body, 0)
    pltpu.make_async_copy(g_hbm.at[sslab_v.at[pl.ds(0, CK)]],
                          rowss[0], semgs[0]).wait()
    pltpu.make_async_copy(dst_hbm.at[pl.ds(0, CK)], dvs[0], semds[0]).wait()
    pltpu.sync_copy(rowss[0], s_sh.at[dvs[0]], add=True)
    plsc.subcore_barrier()
    pltpu.sync_copy(s_sh.at[pl.ds(s * RPT, RPT)],
                    out_hbm.at[pl.ds(c * NP + s * RPT, RPT)])


_scatter_call = pl.kernel(
    _scatter_body,
    out_type=jax.ShapeDtypeStruct((NC * NP, D), jnp.float32),
    mesh=_MESH,
    scratch_types=[
        pltpu.VMEM((EPW,), jnp.int32),
        [pltpu.VMEM((CK,), jnp.int32)] * NBUF,
        [pltpu.VMEM((CK, D), jnp.float32)] * NBUF,
        pltpu.VMEM((ZROWS, D), jnp.float32),
        pltpu.VMEM_SHARED((NP, D), jnp.float32),
        pltpu.SemaphoreType.DMA,
        [pltpu.SemaphoreType.DMA] * NBUF,
        [pltpu.SemaphoreType.DMA] * NBUF,
    ],
)


_TB = 1000


def _tc1_body(x_ref, w_ref, o_ref):
    o_ref[...] = jnp.dot(x_ref[...], w_ref[...],
                         preferred_element_type=jnp.float32)


_tc1_call = pl.pallas_call(
    _tc1_body,
    grid=(N // _TB,),
    in_specs=[
        pl.BlockSpec((_TB, D), lambda i: (i, 0)),
        pl.BlockSpec((D, D), lambda i: (0, 0)),
    ],
    out_specs=pl.BlockSpec((_TB, D), lambda i: (i, 0)),
    out_shape=jax.ShapeDtypeStruct((N, D), jnp.float32),
)


def _tc2_body(sp_ref, g_ref, dinv_ref, b_ref, w_ref, o_ref):
    ssum = sp_ref[0] + sp_ref[1]
    h = jnp.maximum(dinv_ref[...] * (ssum + g_ref[...]) + b_ref[...], 0.0)
    o_ref[...] = jnp.dot(dinv_ref[...] * h, w_ref[...],
                         preferred_element_type=jnp.float32)


_tc2_call = pl.pallas_call(
    _tc2_body,
    grid=(N // _TB,),
    in_specs=[
        pl.BlockSpec((2, _TB, D), lambda i: (0, i, 0)),
        pl.BlockSpec((_TB, D), lambda i: (i, 0)),
        pl.BlockSpec((_TB, 1), lambda i: (i, 0)),
        pl.BlockSpec((1, D), lambda i: (0, 0)),
        pl.BlockSpec((D, D), lambda i: (0, 0)),
    ],
    out_specs=pl.BlockSpec((_TB, D), lambda i: (i, 0)),
    out_shape=jax.ShapeDtypeStruct((N, D), jnp.float32),
)


def _tc3_body(sp_ref, g_ref, dinv_ref, b_ref, o_ref):
    ssum = sp_ref[0] + sp_ref[1]
    o_ref[...] = jnp.maximum(
        dinv_ref[...] * (ssum + g_ref[...]) + b_ref[...], 0.0)


_tc3_call = pl.pallas_call(
    _tc3_body,
    grid=(N // _TB,),
    in_specs=[
        pl.BlockSpec((2, _TB, D), lambda i: (0, i, 0)),
        pl.BlockSpec((_TB, D), lambda i: (i, 0)),
        pl.BlockSpec((_TB, 1), lambda i: (i, 0)),
        pl.BlockSpec((1, D), lambda i: (0, 0)),
    ],
    out_specs=pl.BlockSpec((_TB, D), lambda i: (i, 0)),
    out_shape=jax.ShapeDtypeStruct((N, D), jnp.float32),
)


def kernel(x, edge_index, W1, b1, W2, b2):
    ei = edge_index.astype(jnp.int32)
    src = ei[0]
    dst = ei[1]

    degp = _deg_call(dst)
    u1 = _tc1_call(x, W1)
    deg = 1.0 + degp[:N] + degp[DEGP:DEGP + N]
    dinv = lax.rsqrt(deg)[:, None]
    b1r = b1[None, :]
    b2r = b2[None, :]

    g1 = dinv * u1
    s1 = _scatter_call(g1, src, dst).reshape(NC, NP, D)
    g2 = _tc2_call(s1, g1, dinv, b1r, W2)
    s2 = _scatter_call(g2, src, dst).reshape(NC, NP, D)
    return _tc3_call(s2, g2, dinv, b2r)
